# baseline (speedup 1.0000x reference)
import functools

import jax
import jax.numpy as jnp
from jax import lax
from jax.experimental import pallas as pl
from jax.experimental.pallas import tpu as pltpu
from jax.experimental.pallas import tpu_sc as plsc

NC, NS = 2, 16
NW = NC * NS
DIM = 64
BATCH, SEQ = 4096, 200
ROWS = BATCH * SEQ
RPW = ROWS // NW
CH = 100
NCHUNK = RPW // CH
NJ = DIM // 16


def _rsqrt(a):
    i = lax.bitcast_convert_type(a, jnp.int32)
    i = jnp.int32(0x5F3759DF) - (i >> 1)
    y = lax.bitcast_convert_type(i, jnp.float32)
    y = y * (1.5 - 0.5 * a * y * y)
    y = y * (1.5 - 0.5 * a * y * y)
    return y


_MESH = plsc.VectorSubcoreMesh(
    core_axis_name="c", subcore_axis_name="s", num_cores=NC, num_subcores=NS
)


@functools.partial(
    pl.kernel,
    out_type=jax.ShapeDtypeStruct((NW, NCHUNK, CH, DIM), jnp.float32),
    mesh=_MESH,
    scratch_types=[
        pltpu.VMEM((NCHUNK, CH), jnp.int32),
        pltpu.VMEM((CH, DIM), jnp.float32),
        pltpu.VMEM((CH, DIM), jnp.float32),
        pltpu.VMEM((CH, DIM), jnp.float32),
        pltpu.VMEM((CH, DIM), jnp.float32),
        pltpu.VMEM((SEQ, DIM), jnp.float32),
        pltpu.VMEM((2, DIM), jnp.float32),
        pltpu.SemaphoreType.DMA,
        pltpu.SemaphoreType.DMA,
        pltpu.SemaphoreType.DMA,
        pltpu.SemaphoreType.DMA,
    ],
    compiler_params=pltpu.CompilerParams(
        needs_layout_passes=False, use_tc_tiling_on_sc=False
    ),
)
def _sc_embed_ln(idx_hbm, wt_hbm, pos_hbm, gam_hbm, bet_hbm, out_hbm,
                 idx_v, row0, row1, ob0, ob1, pos_v, gb_v,
                 gsem0, gsem1, osem0, osem1):
    wid = lax.axis_index("s") * NC + lax.axis_index("c")

    pltpu.sync_copy(idx_hbm.at[wid], idx_v)
    pltpu.sync_copy(pos_hbm, pos_v)
    pltpu.sync_copy(gam_hbm, gb_v.at[0])
    pltpu.sync_copy(bet_hbm, gb_v.at[1])

    gvec = [gb_v[0, pl.ds(16 * j, 16)] for j in range(NJ)]
    bvec = [gb_v[1, pl.ds(16 * j, 16)] for j in range(NJ)]
    rows = (row0, row1)
    obs = (ob0, ob1)
    gsems = (gsem0, gsem1)
    osems = (osem0, osem1)

    pltpu.async_copy(wt_hbm.at[idx_v.at[0]], row0, gsem0)
    pltpu.async_copy(wt_hbm.at[idx_v.at[1]], row1, gsem1)

    def compute(b):
        rb, obuf = rows[b], obs[b]
        phase = b * CH

        @pl.loop(0, CH)
        def _(s):
            p = phase + s
            x = [rb[s, pl.ds(16 * j, 16)] + pos_v[p, pl.ds(16 * j, 16)]
                 for j in range(NJ)]
            ssum = (x[0] + x[1]) + (x[2] + x[3])
            qsum = (x[0] * x[0] + x[1] * x[1]) + (x[2] * x[2] + x[3] * x[3])
            tot = jnp.full((16,), jnp.sum(ssum), jnp.float32)
            tot2 = jnp.full((16,), jnp.sum(qsum), jnp.float32)
            mu = tot * (1.0 / DIM)
            var = tot2 * (1.0 / DIM) - mu * mu
            rs = _rsqrt(var + 1e-5)
            for j in range(NJ):
                obuf[s, pl.ds(16 * j, 16)] = (x[j] - mu) * (rs * gvec[j]) + bvec[j]

    def do_chunk(g, b, first, last):
        rb, obuf = rows[b], obs[b]
        pltpu.make_async_copy(wt_hbm.at[idx_v.at[g]], rb, gsems[b]).wait()
        if not first:
            pltpu.make_async_copy(obuf, out_hbm.at[wid, g - 2], osems[b]).wait()
        compute(b)
        if not last:
            pltpu.async_copy(wt_hbm.at[idx_v.at[g + 2]], rb, gsems[b])
        pltpu.async_copy(obuf, out_hbm.at[wid, g], osems[b])

    do_chunk(0, 0, True, False)
    do_chunk(1, 1, True, False)

    @pl.loop(1, NCHUNK // 2 - 1)
    def _(P):
        g = 2 * P
        do_chunk(g, 0, False, False)
        do_chunk(g + 1, 1, False, False)

    do_chunk(NCHUNK - 2, 0, False, True)
    do_chunk(NCHUNK - 1, 1, False, True)
    pltpu.make_async_copy(ob0, out_hbm.at[wid, NCHUNK - 2], osem0).wait()
    pltpu.make_async_copy(ob1, out_hbm.at[wid, NCHUNK - 1], osem1).wait()


def kernel(input, word_table, pos_table, gamma, beta):
    seq = input.shape[-1]
    idx3 = input.reshape(NW, NCHUNK, CH)
    pos2 = pos_table[:seq]
    out = _sc_embed_ln(idx3, word_table, pos2, gamma, beta)
    return out.reshape(BATCH, SEQ, DIM)


# speedup vs baseline: 1.0050x; 1.0050x over previous
import functools

import jax
import jax.numpy as jnp
from jax import lax
from jax.experimental import pallas as pl
from jax.experimental.pallas import tpu as pltpu
from jax.experimental.pallas import tpu_sc as plsc

NC, NS = 2, 16
NW = NC * NS
DIM = 64
BATCH, SEQ = 4096, 200
ROWS = BATCH * SEQ
RPW = ROWS // NW
CH = 100
NCHUNK = RPW // CH
NJ = DIM // 16


def _rsqrt(a):
    i = lax.bitcast_convert_type(a, jnp.int32)
    i = jnp.int32(0x5F3759DF) - (i >> 1)
    y = lax.bitcast_convert_type(i, jnp.float32)
    y = y * (1.5 - 0.5 * a * y * y)
    y = y * (1.5 - 0.5 * a * y * y)
    return y


_MESH = plsc.VectorSubcoreMesh(
    core_axis_name="c", subcore_axis_name="s", num_cores=NC, num_subcores=NS
)


@functools.partial(
    pl.kernel,
    out_type=jax.ShapeDtypeStruct((NW, NCHUNK, CH, DIM), jnp.float32),
    mesh=_MESH,
    scratch_types=[
        pltpu.VMEM((NCHUNK, CH), jnp.int32),
        pltpu.VMEM((CH, DIM), jnp.float32),
        pltpu.VMEM((CH, DIM), jnp.float32),
        pltpu.VMEM((CH, DIM), jnp.float32),
        pltpu.VMEM((CH, DIM), jnp.float32),
        pltpu.VMEM((SEQ, DIM), jnp.float32),
        pltpu.VMEM((2, DIM), jnp.float32),
        pltpu.SemaphoreType.DMA,
        pltpu.SemaphoreType.DMA,
        pltpu.SemaphoreType.DMA,
        pltpu.SemaphoreType.DMA,
    ],
    compiler_params=pltpu.CompilerParams(
        needs_layout_passes=False, use_tc_tiling_on_sc=False
    ),
)
def _sc_embed_ln(idx_hbm, wt_hbm, pos_hbm, gam_hbm, bet_hbm, out_hbm,
                 idx_v, row0, row1, ob0, ob1, pos_v, gb_v,
                 gsem0, gsem1, osem0, osem1):
    wid = lax.axis_index("s") * NC + lax.axis_index("c")

    pltpu.sync_copy(idx_hbm.at[wid], idx_v)
    pltpu.sync_copy(pos_hbm, pos_v)
    pltpu.sync_copy(gam_hbm, gb_v.at[0])
    pltpu.sync_copy(bet_hbm, gb_v.at[1])

    gvec = [gb_v[0, pl.ds(16 * j, 16)] for j in range(NJ)]
    bvec = [gb_v[1, pl.ds(16 * j, 16)] for j in range(NJ)]
    rows = (row0, row1)
    obs = (ob0, ob1)
    gsems = (gsem0, gsem1)
    osems = (osem0, osem1)

    pltpu.async_copy(wt_hbm.at[idx_v.at[0]], row0, gsem0)
    pltpu.async_copy(wt_hbm.at[idx_v.at[1]], row1, gsem1)

    def compute(b):
        rb, obuf = rows[b], obs[b]
        phase = b * CH

        @plsc.parallel_loop(0, CH, unroll=4)
        def _(s):
            p = phase + s
            x = [rb[s, pl.ds(16 * j, 16)] + pos_v[p, pl.ds(16 * j, 16)]
                 for j in range(NJ)]
            ssum = (x[0] + x[1]) + (x[2] + x[3])
            qsum = (x[0] * x[0] + x[1] * x[1]) + (x[2] * x[2] + x[3] * x[3])
            tot = jnp.full((16,), jnp.sum(ssum), jnp.float32)
            tot2 = jnp.full((16,), jnp.sum(qsum), jnp.float32)
            mu = tot * (1.0 / DIM)
            var = tot2 * (1.0 / DIM) - mu * mu
            rs = _rsqrt(var + 1e-5)
            for j in range(NJ):
                obuf[s, pl.ds(16 * j, 16)] = (x[j] - mu) * (rs * gvec[j]) + bvec[j]

    def do_chunk(g, b, first, last):
        rb, obuf = rows[b], obs[b]
        pltpu.make_async_copy(wt_hbm.at[idx_v.at[g]], rb, gsems[b]).wait()
        if not first:
            pltpu.make_async_copy(obuf, out_hbm.at[wid, g - 2], osems[b]).wait()
        compute(b)
        if not last:
            pltpu.async_copy(wt_hbm.at[idx_v.at[g + 2]], rb, gsems[b])
        pltpu.async_copy(obuf, out_hbm.at[wid, g], osems[b])

    do_chunk(0, 0, True, False)
    do_chunk(1, 1, True, False)

    @pl.loop(1, NCHUNK // 2 - 1)
    def _(P):
        g = 2 * P
        do_chunk(g, 0, False, False)
        do_chunk(g + 1, 1, False, False)

    do_chunk(NCHUNK - 2, 0, False, True)
    do_chunk(NCHUNK - 1, 1, False, True)
    pltpu.make_async_copy(ob0, out_hbm.at[wid, NCHUNK - 2], osem0).wait()
    pltpu.make_async_copy(ob1, out_hbm.at[wid, NCHUNK - 1], osem1).wait()


def kernel(input, word_table, pos_table, gamma, beta):
    seq = input.shape[-1]
    idx3 = input.reshape(NW, NCHUNK, CH)
    pos2 = pos_table[:seq]
    out = _sc_embed_ln(idx3, word_table, pos2, gamma, beta)
    return out.reshape(BATCH, SEQ, DIM)


# speedup vs baseline: 1.0478x; 1.0426x over previous
import functools

import jax
import jax.numpy as jnp
from jax import lax
from jax.experimental import pallas as pl
from jax.experimental.pallas import tpu as pltpu
from jax.experimental.pallas import tpu_sc as plsc

NC, NS = 2, 16
NW = NC * NS
VOC = 1000000
DIM = 64
BATCH, SEQ = 4096, 200
ROWS = BATCH * SEQ
RPW = ROWS // NW
CH = 100
NCHUNK = RPW // CH
BPW = BATCH // NW
NJ = DIM // 16


def _rsqrt(a):
    i = lax.bitcast_convert_type(a, jnp.int32)
    i = jnp.int32(0x5F3759DF) - (i >> 1)
    y = lax.bitcast_convert_type(i, jnp.float32)
    y = y * (1.5 - 0.5 * a * y * y)
    y = y * (1.5 - 0.5 * a * y * y)
    return y


_MESH = plsc.VectorSubcoreMesh(
    core_axis_name="c", subcore_axis_name="s", num_cores=NC, num_subcores=NS
)


@functools.partial(
    pl.kernel,
    out_type=jax.ShapeDtypeStruct((BATCH, SEQ, DIM), jnp.float32),
    mesh=_MESH,
    scratch_types=[
        pltpu.VMEM((NCHUNK, CH), jnp.int32),
        pltpu.VMEM((CH, DIM), jnp.float32),
        pltpu.VMEM((CH, DIM), jnp.float32),
        pltpu.VMEM((CH, DIM), jnp.float32),
        pltpu.VMEM((CH, DIM), jnp.float32),
        pltpu.VMEM((SEQ, DIM), jnp.float32),
        pltpu.VMEM((2, DIM), jnp.float32),
        pltpu.SemaphoreType.DMA,
        pltpu.SemaphoreType.DMA,
        pltpu.SemaphoreType.DMA,
        pltpu.SemaphoreType.DMA,
    ],
    compiler_params=pltpu.CompilerParams(
        needs_layout_passes=False, use_tc_tiling_on_sc=False
    ),
)
def _sc_embed_ln(idx_hbm, wt_hbm, pos_hbm, gam_hbm, bet_hbm, out_hbm,
                 idx_v, row0, row1, ob0, ob1, pos_v, gb_v,
                 gsem0, gsem1, osem0, osem1):
    wid = lax.axis_index("s") * NC + lax.axis_index("c")

    pltpu.sync_copy(idx_hbm.at[wid], idx_v)
    pltpu.sync_copy(pos_hbm, pos_v)
    pltpu.sync_copy(gam_hbm, gb_v.at[0])
    pltpu.sync_copy(bet_hbm, gb_v.at[1])

    gvec = [gb_v[0, pl.ds(16 * j, 16)] for j in range(NJ)]
    bvec = [gb_v[1, pl.ds(16 * j, 16)] for j in range(NJ)]
    rows = (row0, row1)
    obs = (ob0, ob1)
    gsems = (gsem0, gsem1)
    osems = (osem0, osem1)

    def gather_src(g):
        return wt_hbm.at[idx_v.at[g]]

    pltpu.async_copy(gather_src(0), row0, gsem0)
    pltpu.async_copy(gather_src(1), row1, gsem1)

    def compute(b):
        rb, obuf = rows[b], obs[b]
        phase = b * CH

        @plsc.parallel_loop(0, CH, unroll=4)
        def _(s):
            p = phase + s
            x = [rb[s, pl.ds(16 * j, 16)] + pos_v[p, pl.ds(16 * j, 16)]
                 for j in range(NJ)]
            ssum = (x[0] + x[1]) + (x[2] + x[3])
            qsum = (x[0] * x[0] + x[1] * x[1]) + (x[2] * x[2] + x[3] * x[3])
            tot = jnp.full((16,), jnp.sum(ssum), jnp.float32)
            tot2 = jnp.full((16,), jnp.sum(qsum), jnp.float32)
            mu = tot * (1.0 / DIM)
            var = tot2 * (1.0 / DIM) - mu * mu
            rs = _rsqrt(var + 1e-5)
            for j in range(NJ):
                obuf[s, pl.ds(16 * j, 16)] = (x[j] - mu) * (rs * gvec[j]) + bvec[j]

    def do_chunk(P, b, first, last):
        g = 2 * P + b
        bi = wid * BPW + P
        rb, obuf = rows[b], obs[b]
        pltpu.make_async_copy(gather_src(g), rb, gsems[b]).wait()
        if not first:
            pltpu.make_async_copy(
                obuf, out_hbm.at[bi - 1, pl.ds(b * CH, CH)], osems[b]
            ).wait()
        compute(b)
        if not last:
            pltpu.async_copy(gather_src(g + 2), rb, gsems[b])
        pltpu.async_copy(obuf, out_hbm.at[bi, pl.ds(b * CH, CH)], osems[b])

    do_chunk(0, 0, True, False)
    do_chunk(0, 1, True, False)

    @pl.loop(1, NCHUNK // 2 - 1)
    def _(P):
        do_chunk(P, 0, False, False)
        do_chunk(P, 1, False, False)

    do_chunk(NCHUNK // 2 - 1, 0, False, True)
    do_chunk(NCHUNK // 2 - 1, 1, False, True)
    last_bi = wid * BPW + BPW - 1
    pltpu.make_async_copy(ob0, out_hbm.at[last_bi, pl.ds(0, CH)], osem0).wait()
    pltpu.make_async_copy(ob1, out_hbm.at[last_bi, pl.ds(CH, CH)], osem1).wait()


def kernel(input, word_table, pos_table, gamma, beta):
    seq = input.shape[-1]
    idx2 = (input * 2).reshape(NW, NCHUNK, CH)
    wt2 = jnp.pad(word_table, ((0, 0), (0, 128 - DIM))).reshape(2 * VOC, DIM)
    pos2 = pos_table[:seq]
    return _sc_embed_ln(idx2, wt2, pos2, gamma, beta)
